# baseline (speedup 1.0000x reference)
import functools

import jax
import jax.numpy as jnp
from jax import lax
from jax.experimental import pallas as pl
from jax.experimental.pallas import tpu as pltpu
from jax.experimental.pallas import tpu_sc as plsc

_N, _D, _E = 10000, 128, 320000
_NC, _NS = 2, 16
_HD = _D // _NC
_PACKW = 128
_CHUNK = 64
_NCHUNK_W = 160
_NBUF = 4
_NG = 2 * _NCHUNK_W // _NBUF
_EPAD = _NS * _NCHUNK_W * _PACKW
_NROWS = 10240
_RPS = _NROWS // _NS


def _gather_add_body(x_hbm, pack_hbm, zeros_hbm, out_hbm,
                     packv, colbM, rowbM, gbuf, acc, sem0, sem1, sem2, sem3):
    cid = lax.axis_index("c")
    sid = lax.axis_index("s")

    pltpu.sync_copy(pack_hbm.at[sid], packv)
    pltpu.sync_copy(zeros_hbm, acc.at[pl.ds(sid * _RPS, _RPS)])
    plsc.subcore_barrier()

    sems = (sem0, sem1, sem2, sem3)

    def unpack(j, h, q):
        for k in range(_CHUNK // 16):
            v = packv[j, pl.ds(h * _CHUNK + k * 16, 16)]
            colbM[q, pl.ds(k * 16, 16)] = ((v & 0xFFFF) << 1) | cid
            rowbM[q, pl.ds(k * 16, 16)] = v >> 16

    for q in range(_NBUF):
        unpack(q // 2, q % 2, q)
        pltpu.async_copy(x_hbm.at[colbM.at[q]], gbuf.at[q], sems[q])

    def step(g, carry):
        for q in range(_NBUF):
            pltpu.make_async_copy(x_hbm.at[colbM.at[q]], gbuf.at[q],
                                  sems[q]).wait()
            pltpu.sync_copy(gbuf.at[q], acc.at[rowbM.at[q]], add=True)

            @pl.when(g + 1 < _NG)
            def _():
                c = _NBUF * (g + 1) + q
                unpack(c // 2, q % 2, q)
                pltpu.async_copy(x_hbm.at[colbM.at[q]], gbuf.at[q], sems[q])
        return carry

    lax.fori_loop(0, _NG, step, 0)
    plsc.subcore_barrier()

    pltpu.sync_copy(acc.at[pl.ds(sid * _RPS, _RPS)],
                    out_hbm.at[pl.ds(cid * _NROWS + sid * _RPS, _RPS)])


_ga_kernel = functools.partial(
    pl.kernel,
    out_type=jax.ShapeDtypeStruct((_NC * _NROWS, _HD), jnp.float32),
    mesh=plsc.VectorSubcoreMesh(core_axis_name="c", subcore_axis_name="s",
                                num_cores=_NC, num_subcores=_NS),
    compiler_params=pltpu.CompilerParams(use_tc_tiling_on_sc=False),
    scratch_types=[
        pltpu.VMEM((_NCHUNK_W, _PACKW), jnp.int32),
        pltpu.VMEM((_NBUF, _CHUNK), jnp.int32),
        pltpu.VMEM((_NBUF, _CHUNK), jnp.int32),
        pltpu.VMEM((_NBUF, _CHUNK, _HD), jnp.float32),
        pltpu.VMEM_SHARED((_NROWS, _HD), jnp.float32),
        pltpu.SemaphoreType.DMA,
        pltpu.SemaphoreType.DMA,
        pltpu.SemaphoreType.DMA,
        pltpu.SemaphoreType.DMA,
    ],
)(_gather_add_body)


def kernel(r0, edge_index, train_mask):
    row = edge_index[0]
    col = edge_index[1]
    pad = _EPAD - _E
    rowp = jnp.concatenate([row, jnp.full((pad,), _N, jnp.int32)])
    colp = jnp.concatenate([col, jnp.zeros((pad,), jnp.int32)])
    packp = ((rowp << 16) | colp).reshape(_NS, _NCHUNK_W, _PACKW)
    zeros = jnp.zeros((_RPS, _HD), jnp.float32)

    def S(x):
        p = _ga_kernel(x.reshape(_NC * _N, _HD), packp, zeros)
        return jnp.concatenate([p[:_N], p[_NROWS:_NROWS + _N]], axis=1)

    deg = S(jnp.ones((_N, _D), jnp.float32))[:, 0]
    dinv = jnp.where(deg > 0, 1.0 / jnp.sqrt(jnp.maximum(deg, 1e-12)), 0.0)
    maskf = train_mask.astype(jnp.float32)
    in_scale = (maskf * dinv)[:, None]
    mid_scale = (dinv * dinv)[:, None]
    out_scale = (-dinv)[:, None]

    def apply_L(r):
        z = S(in_scale * r)
        z = S(mid_scale * z)
        return out_scale * z

    dt = 0.2
    out = [r0]
    r = r0
    for _ in range(5):
        s1 = apply_L(r)
        s2 = apply_L(r + 0.5 * dt * s1)
        s3 = apply_L(r + 0.5 * dt * s2)
        s4 = apply_L(r + dt * s3)
        r = r + dt / 6.0 * (s1 + 2.0 * s2 + 2.0 * s3 + s4)
        out.append(r)
    return jnp.stack(out, axis=0)
